# baseline (speedup 1.0000x reference)
import functools

import jax
import jax.numpy as jnp
from jax import lax
from jax.experimental import pallas as pl
from jax.experimental.pallas import tpu as pltpu
from jax.experimental.pallas import tpu_sc as plsc

N = 10000
E = 320000
D = 128
BN_EPS = 1e-5

NC = 2
NS = 16
NW = NC * NS
L = 16

EP = E // NW
CH = 100
NCH = EP // CH
RT = N // NS
RB = 125
NRB = RT // RB
NSET = 3
BD = jnp.bfloat16
LB = 32

_mesh = functools.partial(
    plsc.VectorSubcoreMesh, core_axis_name="c", subcore_axis_name="s",
    num_cores=NC, num_subcores=NS)
_sc_params = pltpu.CompilerParams(
    needs_layout_passes=False, use_tc_tiling_on_sc=False)


@functools.partial(
    pl.kernel,
    out_type=(jax.ShapeDtypeStruct((NW, N), jnp.float32),
              jax.ShapeDtypeStruct((NW, N), jnp.float32)),
    mesh=_mesh(),
    compiler_params=_sc_params,
    scratch_types=[
        pltpu.VMEM((EP // L, L), jnp.int32),
        pltpu.VMEM((EP // L, L), jnp.int32),
        pltpu.VMEM((N,), jnp.float32),
        pltpu.VMEM((N,), jnp.float32),
    ],
)
def _deg_kernel(src_hbm, dst_hbm, degs_hbm, degd_hbm,
                src_v, dst_v, degs_v, degd_v):
    c = lax.axis_index("c")
    s = lax.axis_index("s")
    wid = s * NC + c
    pltpu.sync_copy(src_hbm.at[wid], src_v)
    pltpu.sync_copy(dst_hbm.at[wid], dst_v)

    z = jnp.zeros((L,), jnp.float32)

    @pl.loop(0, N // L)
    def _zero(i):
        degs_v[pl.ds(i * L, L)] = z
        degd_v[pl.ds(i * L, L)] = z

    ones = jnp.ones((L,), jnp.float32)

    @pl.loop(0, EP // L)
    def _acc(j):
        plsc.addupdate_scatter(degs_v, [src_v[j]], ones)
        plsc.addupdate_scatter(degd_v, [dst_v[j]], ones)

    pltpu.sync_copy(degs_v, degs_hbm.at[wid])
    pltpu.sync_copy(degd_v, degd_hbm.at[wid])


@functools.partial(
    pl.kernel,
    out_type=jax.ShapeDtypeStruct((NC, N, D), BD),
    mesh=_mesh(),
    compiler_params=_sc_params,
    scratch_types=[
        pltpu.VMEM((NSET, 2, CH), jnp.int32),
        pltpu.VMEM((NSET, CH, D), BD),
        pltpu.VMEM_SHARED((N, D), BD),
        pltpu.SemaphoreType.DMA,
        pltpu.SemaphoreType.DMA,
        pltpu.SemaphoreType.DMA,
    ],
)
def _spmm_kernel(h_hbm, eidx_hbm, agg_hbm,
                 pidx, rows_v, agg_sh, gsem, ssem, isem):
    c = lax.axis_index("c")
    s = lax.axis_index("s")
    wid = s * NC + c

    z = jnp.zeros((LB,), BD)

    @pl.loop(0, CH)
    def _zb(j):
        for k in range(D // LB):
            rows_v[0, j, pl.ds(k * LB, LB)] = z

    row0 = s * RT
    nz = RT // CH
    for t in range(nz):
        pltpu.async_copy(rows_v.at[0], agg_sh.at[pl.ds(row0 + t * CH, CH)],
                         ssem)
    rz = RT - nz * CH
    pltpu.async_copy(rows_v.at[0, pl.ds(0, rz)],
                     agg_sh.at[pl.ds(row0 + nz * CH, rz)], ssem)
    for t in range(nz):
        pltpu.make_async_copy(rows_v.at[0], agg_sh.at[pl.ds(0, CH)],
                              ssem).wait()
    pltpu.make_async_copy(rows_v.at[0, pl.ds(0, rz)],
                          agg_sh.at[pl.ds(0, rz)], ssem).wait()
    plsc.subcore_barrier()

    def _fire_idx(x, st):
        pltpu.async_copy(eidx_hbm.at[wid, x], pidx.at[st], isem)

    def _drain_idx(st):
        pltpu.make_async_copy(eidx_hbm.at[wid, 0], pidx.at[st], isem).wait()

    def _fire_g(st):
        pltpu.async_copy(h_hbm.at[pidx.at[st, 0]], rows_v.at[st], gsem)

    def _fire_s(st):
        pltpu.async_copy(rows_v.at[st], agg_sh.at[pidx.at[st, 1]], ssem,
                         add=True)

    def _drain_rows(sem, st):
        pltpu.make_async_copy(h_hbm.at[pidx.at[0, 0]], rows_v.at[st],
                              sem).wait()

    _fire_idx(0, 0)
    _drain_idx(0)
    _fire_g(0)
    _fire_idx(1, 1)

    @pl.loop(0, NCH)
    def _grp(x):
        st = lax.rem(x, NSET)
        sn = lax.rem(x + 1, NSET)
        sp = lax.rem(x + 2, NSET)

        @pl.when(x > 0)
        def _():
            _drain_rows(ssem, sp)

        @pl.when(x + 2 < NCH)
        def _():
            _fire_idx(x + 2, sp)

        @pl.when(x + 1 < NCH)
        def _():
            _drain_idx(sn)
            _fire_g(sn)

        _drain_rows(gsem, st)
        _fire_s(st)

    _drain_rows(ssem, lax.rem(jnp.int32(NCH - 1), NSET))
    plsc.subcore_barrier()

    for t in range(NRB):
        r = row0 + t * RB
        pltpu.sync_copy(agg_sh.at[pl.ds(r, RB)], agg_hbm.at[c, pl.ds(r, RB)])


def _norm_body(x_ref, degs_ref, degd_ref, hs_ref, ns_ref, nd_ref):
    deg_out = jnp.sum(degs_ref[...], axis=1, keepdims=True)
    deg_in = jnp.sum(degd_ref[...], axis=1, keepdims=True)
    ns = jnp.where(deg_out > 0, lax.rsqrt(jnp.maximum(deg_out, 1.0)), 0.0)
    nd = jnp.where(deg_in > 0, lax.rsqrt(jnp.maximum(deg_in, 1.0)), 0.0)
    ns_ref[...] = ns
    nd_ref[...] = nd
    hs_ref[...] = (x_ref[...] * ns).astype(BD)


_norm_call = pl.pallas_call(
    _norm_body,
    out_shape=(jax.ShapeDtypeStruct((N, D), BD),
               jax.ShapeDtypeStruct((N, 1), jnp.float32),
               jax.ShapeDtypeStruct((N, 1), jnp.float32)),
)


def _layer_body(relu_and_prescale, aggp_ref, nd_ref, ns_ref, w_ref, b_ref,
                out_ref):
    agg = (aggp_ref[0].astype(jnp.float32) +
           aggp_ref[1].astype(jnp.float32))
    h = agg * nd_ref[...]
    y = jnp.dot(h, w_ref[...], preferred_element_type=jnp.float32)
    y = y + b_ref[...]
    if relu_and_prescale:
        y = jnp.maximum(y, 0.0) * ns_ref[...]
        out_ref[...] = y.astype(BD)
    else:
        out_ref[...] = y


_layer_mid = pl.pallas_call(
    functools.partial(_layer_body, True),
    out_shape=jax.ShapeDtypeStruct((N, D), BD),
)
_layer_last = pl.pallas_call(
    functools.partial(_layer_body, False),
    out_shape=jax.ShapeDtypeStruct((N, D), jnp.float32),
)


def kernel(x, edge_index, W1, b1, g1, be1, W2, b2, g2, be2, W3, b3):
    eidx = edge_index.reshape(2, NW, NCH, CH).transpose(1, 2, 0, 3)
    src_d = edge_index[0].reshape(NW, EP // L, L)
    dst_d = edge_index[1].reshape(NW, EP // L, L)

    degs_p, degd_p = _deg_kernel(src_d, dst_d)
    hs, ns, nd = _norm_call(x, degs_p.T, degd_p.T)

    sc = 1.0 / jnp.sqrt(jnp.float32(1.0) + BN_EPS)
    Wf1 = W1 * (g1 * sc)[None, :]
    bf1 = (b1 * g1 * sc + be1).reshape(1, D)
    Wf2 = W2 * (g2 * sc)[None, :]
    bf2 = (b2 * g2 * sc + be2).reshape(1, D)
    bf3 = b3.reshape(1, D)

    aggp = _spmm_kernel(hs, eidx)
    hs = _layer_mid(aggp, nd, ns, Wf1, bf1)
    aggp = _spmm_kernel(hs, eidx)
    hs = _layer_mid(aggp, nd, ns, Wf2, bf2)
    aggp = _spmm_kernel(hs, eidx)
    out = _layer_last(aggp, nd, ns, W3, bf3)
    return out


# speedup vs baseline: 1.2118x; 1.2118x over previous
import functools

import jax
import jax.numpy as jnp
from jax import lax
from jax.experimental import pallas as pl
from jax.experimental.pallas import tpu as pltpu
from jax.experimental.pallas import tpu_sc as plsc

N = 10000
E = 320000
D = 128
BN_EPS = 1e-5

NC = 2
NS = 16
NW = NC * NS
L = 16

EP = E // NW
CH = 100
NCH = EP // CH
RT = N // NS
RB = 125
NRB = RT // RB
NSET = 6
BD = jnp.bfloat16
LB = 32

_mesh = functools.partial(
    plsc.VectorSubcoreMesh, core_axis_name="c", subcore_axis_name="s",
    num_cores=NC, num_subcores=NS)
_sc_params = pltpu.CompilerParams(
    needs_layout_passes=False, use_tc_tiling_on_sc=False)


@functools.partial(
    pl.kernel,
    out_type=(jax.ShapeDtypeStruct((NW, N), jnp.float32),
              jax.ShapeDtypeStruct((NW, N), jnp.float32)),
    mesh=_mesh(),
    compiler_params=_sc_params,
    scratch_types=[
        pltpu.VMEM((EP // L, L), jnp.int32),
        pltpu.VMEM((EP // L, L), jnp.int32),
        pltpu.VMEM((N,), jnp.float32),
        pltpu.VMEM((N,), jnp.float32),
    ],
)
def _deg_kernel(src_hbm, dst_hbm, degs_hbm, degd_hbm,
                src_v, dst_v, degs_v, degd_v):
    c = lax.axis_index("c")
    s = lax.axis_index("s")
    wid = s * NC + c
    pltpu.sync_copy(src_hbm.at[wid], src_v)
    pltpu.sync_copy(dst_hbm.at[wid], dst_v)

    z = jnp.zeros((L,), jnp.float32)

    @pl.loop(0, N // L)
    def _zero(i):
        degs_v[pl.ds(i * L, L)] = z
        degd_v[pl.ds(i * L, L)] = z

    ones = jnp.ones((L,), jnp.float32)

    @pl.loop(0, EP // L)
    def _acc(j):
        plsc.addupdate_scatter(degs_v, [src_v[j]], ones)
        plsc.addupdate_scatter(degd_v, [dst_v[j]], ones)

    pltpu.sync_copy(degs_v, degs_hbm.at[wid])
    pltpu.sync_copy(degd_v, degd_hbm.at[wid])


@functools.partial(
    pl.kernel,
    out_type=jax.ShapeDtypeStruct((NC, N, D), BD),
    mesh=_mesh(),
    compiler_params=_sc_params,
    scratch_types=[
        pltpu.VMEM((NSET, 2, CH), jnp.int32),
        pltpu.VMEM((NSET, CH, D), BD),
        pltpu.VMEM_SHARED((N, D), BD),
        pltpu.SemaphoreType.DMA,
        pltpu.SemaphoreType.DMA,
        pltpu.SemaphoreType.DMA,
    ],
)
def _spmm_kernel(h_hbm, eidx_hbm, agg_hbm,
                 pidx, rows_v, agg_sh, gsem, ssem, isem):
    c = lax.axis_index("c")
    s = lax.axis_index("s")
    wid = s * NC + c

    z = jnp.zeros((LB,), BD)

    @pl.loop(0, CH)
    def _zb(j):
        for k in range(D // LB):
            rows_v[0, j, pl.ds(k * LB, LB)] = z

    row0 = s * RT
    nz = RT // CH
    for t in range(nz):
        pltpu.async_copy(rows_v.at[0], agg_sh.at[pl.ds(row0 + t * CH, CH)],
                         ssem)
    rz = RT - nz * CH
    pltpu.async_copy(rows_v.at[0, pl.ds(0, rz)],
                     agg_sh.at[pl.ds(row0 + nz * CH, rz)], ssem)
    for t in range(nz):
        pltpu.make_async_copy(rows_v.at[0], agg_sh.at[pl.ds(0, CH)],
                              ssem).wait()
    pltpu.make_async_copy(rows_v.at[0, pl.ds(0, rz)],
                          agg_sh.at[pl.ds(0, rz)], ssem).wait()
    plsc.subcore_barrier()

    def _fire_idx(x, st):
        pltpu.async_copy(eidx_hbm.at[wid, x], pidx.at[st], isem)

    def _drain_idx(st):
        pltpu.make_async_copy(eidx_hbm.at[wid, 0], pidx.at[st], isem).wait()

    def _fire_g(st):
        pltpu.async_copy(h_hbm.at[pidx.at[st, 0]], rows_v.at[st], gsem)

    def _fire_s(st):
        pltpu.async_copy(rows_v.at[st], agg_sh.at[pidx.at[st, 1]], ssem,
                         add=True)

    def _drain_rows(sem, st):
        pltpu.make_async_copy(h_hbm.at[pidx.at[0, 0]], rows_v.at[st],
                              sem).wait()

    _fire_idx(0, 0)
    _fire_idx(1, 1)
    _drain_idx(0)
    _fire_g(0)
    _drain_idx(1)
    _fire_g(1)
    _fire_idx(2, 2)
    _fire_idx(3, 3)

    @pl.loop(0, NCH)
    def _grp(x):
        st = lax.rem(x, NSET)
        s2 = lax.rem(x + 2, NSET)
        s4 = lax.rem(x + 4, NSET)

        @pl.when(x > 1)
        def _():
            _drain_rows(ssem, s4)

        @pl.when(x + 4 < NCH)
        def _():
            _fire_idx(x + 4, s4)

        @pl.when(x + 2 < NCH)
        def _():
            _drain_idx(s2)
            _fire_g(s2)

        _drain_rows(gsem, st)
        _fire_s(st)

    _drain_rows(ssem, lax.rem(jnp.int32(NCH - 2), NSET))
    _drain_rows(ssem, lax.rem(jnp.int32(NCH - 1), NSET))
    plsc.subcore_barrier()

    for t in range(NRB):
        r = row0 + t * RB
        pltpu.sync_copy(agg_sh.at[pl.ds(r, RB)], agg_hbm.at[c, pl.ds(r, RB)])


def _norm_body(x_ref, degs_ref, degd_ref, hs_ref, ns_ref, nd_ref):
    deg_out = jnp.sum(degs_ref[...], axis=1, keepdims=True)
    deg_in = jnp.sum(degd_ref[...], axis=1, keepdims=True)
    ns = jnp.where(deg_out > 0, lax.rsqrt(jnp.maximum(deg_out, 1.0)), 0.0)
    nd = jnp.where(deg_in > 0, lax.rsqrt(jnp.maximum(deg_in, 1.0)), 0.0)
    ns_ref[...] = ns
    nd_ref[...] = nd
    hs_ref[...] = (x_ref[...] * ns).astype(BD)


_norm_call = pl.pallas_call(
    _norm_body,
    out_shape=(jax.ShapeDtypeStruct((N, D), BD),
               jax.ShapeDtypeStruct((N, 1), jnp.float32),
               jax.ShapeDtypeStruct((N, 1), jnp.float32)),
)


def _layer_body(relu_and_prescale, aggp_ref, nd_ref, ns_ref, w_ref, b_ref,
                out_ref):
    agg = (aggp_ref[0].astype(jnp.float32) +
           aggp_ref[1].astype(jnp.float32))
    h = agg * nd_ref[...]
    y = jnp.dot(h, w_ref[...], preferred_element_type=jnp.float32)
    y = y + b_ref[...]
    if relu_and_prescale:
        y = jnp.maximum(y, 0.0) * ns_ref[...]
        out_ref[...] = y.astype(BD)
    else:
        out_ref[...] = y


_layer_mid = pl.pallas_call(
    functools.partial(_layer_body, True),
    out_shape=jax.ShapeDtypeStruct((N, D), BD),
)
_layer_last = pl.pallas_call(
    functools.partial(_layer_body, False),
    out_shape=jax.ShapeDtypeStruct((N, D), jnp.float32),
)


def kernel(x, edge_index, W1, b1, g1, be1, W2, b2, g2, be2, W3, b3):
    eidx = edge_index.reshape(2, NW, NCH, CH).transpose(1, 2, 0, 3)
    src_d = edge_index[0].reshape(NW, EP // L, L)
    dst_d = edge_index[1].reshape(NW, EP // L, L)

    degs_p, degd_p = _deg_kernel(src_d, dst_d)
    hs, ns, nd = _norm_call(x, degs_p.T, degd_p.T)

    sc = 1.0 / jnp.sqrt(jnp.float32(1.0) + BN_EPS)
    Wf1 = W1 * (g1 * sc)[None, :]
    bf1 = (b1 * g1 * sc + be1).reshape(1, D)
    Wf2 = W2 * (g2 * sc)[None, :]
    bf2 = (b2 * g2 * sc + be2).reshape(1, D)
    bf3 = b3.reshape(1, D)

    aggp = _spmm_kernel(hs, eidx)
    hs = _layer_mid(aggp, nd, ns, Wf1, bf1)
    aggp = _spmm_kernel(hs, eidx)
    hs = _layer_mid(aggp, nd, ns, Wf2, bf2)
    aggp = _spmm_kernel(hs, eidx)
    out = _layer_last(aggp, nd, ns, W3, bf3)
    return out


# speedup vs baseline: 1.2301x; 1.0151x over previous
import functools

import jax
import jax.numpy as jnp
from jax import lax
from jax.experimental import pallas as pl
from jax.experimental.pallas import tpu as pltpu
from jax.experimental.pallas import tpu_sc as plsc

N = 10000
E = 320000
D = 128
BN_EPS = 1e-5

NC = 2
NS = 16
NW = NC * NS
L = 16

EP = E // NW
CH = 100
NCH = EP // CH
RT = N // NS
RB = 125
NRB = RT // RB
DQ = 3
NSET = 3 * DQ
BD = jnp.bfloat16
LB = 32

_mesh = functools.partial(
    plsc.VectorSubcoreMesh, core_axis_name="c", subcore_axis_name="s",
    num_cores=NC, num_subcores=NS)
_sc_params = pltpu.CompilerParams(
    needs_layout_passes=False, use_tc_tiling_on_sc=False)


@functools.partial(
    pl.kernel,
    out_type=(jax.ShapeDtypeStruct((NW, N), jnp.float32),
              jax.ShapeDtypeStruct((NW, N), jnp.float32)),
    mesh=_mesh(),
    compiler_params=_sc_params,
    scratch_types=[
        pltpu.VMEM((EP // L, L), jnp.int32),
        pltpu.VMEM((EP // L, L), jnp.int32),
        pltpu.VMEM((N,), jnp.float32),
        pltpu.VMEM((N,), jnp.float32),
    ],
)
def _deg_kernel(src_hbm, dst_hbm, degs_hbm, degd_hbm,
                src_v, dst_v, degs_v, degd_v):
    c = lax.axis_index("c")
    s = lax.axis_index("s")
    wid = s * NC + c
    pltpu.sync_copy(src_hbm.at[wid], src_v)
    pltpu.sync_copy(dst_hbm.at[wid], dst_v)

    z = jnp.zeros((L,), jnp.float32)

    @pl.loop(0, N // L)
    def _zero(i):
        degs_v[pl.ds(i * L, L)] = z
        degd_v[pl.ds(i * L, L)] = z

    ones = jnp.ones((L,), jnp.float32)

    @pl.loop(0, EP // L)
    def _acc(j):
        plsc.addupdate_scatter(degs_v, [src_v[j]], ones)
        plsc.addupdate_scatter(degd_v, [dst_v[j]], ones)

    pltpu.sync_copy(degs_v, degs_hbm.at[wid])
    pltpu.sync_copy(degd_v, degd_hbm.at[wid])


@functools.partial(
    pl.kernel,
    out_type=jax.ShapeDtypeStruct((NC, N, D), BD),
    mesh=_mesh(),
    compiler_params=_sc_params,
    scratch_types=[
        pltpu.VMEM((NSET, 2, CH), jnp.int32),
        pltpu.VMEM((NSET, CH, D), BD),
        pltpu.VMEM_SHARED((N, D), BD),
        pltpu.SemaphoreType.DMA,
        pltpu.SemaphoreType.DMA,
        pltpu.SemaphoreType.DMA,
    ],
)
def _spmm_kernel(h_hbm, eidx_hbm, agg_hbm,
                 pidx, rows_v, agg_sh, gsem, ssem, isem):
    c = lax.axis_index("c")
    s = lax.axis_index("s")
    wid = s * NC + c

    z = jnp.zeros((LB,), BD)

    @pl.loop(0, CH)
    def _zb(j):
        for k in range(D // LB):
            rows_v[0, j, pl.ds(k * LB, LB)] = z

    row0 = s * RT
    nz = RT // CH
    for t in range(nz):
        pltpu.async_copy(rows_v.at[0], agg_sh.at[pl.ds(row0 + t * CH, CH)],
                         ssem)
    rz = RT - nz * CH
    pltpu.async_copy(rows_v.at[0, pl.ds(0, rz)],
                     agg_sh.at[pl.ds(row0 + nz * CH, rz)], ssem)
    for t in range(nz):
        pltpu.make_async_copy(rows_v.at[0], agg_sh.at[pl.ds(0, CH)],
                              ssem).wait()
    pltpu.make_async_copy(rows_v.at[0, pl.ds(0, rz)],
                          agg_sh.at[pl.ds(0, rz)], ssem).wait()
    plsc.subcore_barrier()

    def _fire_idx(x, st):
        pltpu.async_copy(eidx_hbm.at[wid, x], pidx.at[st], isem)

    def _drain_idx(st):
        pltpu.make_async_copy(eidx_hbm.at[wid, 0], pidx.at[st], isem).wait()

    def _fire_g(st):
        pltpu.async_copy(h_hbm.at[pidx.at[st, 0]], rows_v.at[st], gsem)

    def _fire_s(st):
        pltpu.async_copy(rows_v.at[st], agg_sh.at[pidx.at[st, 1]], ssem,
                         add=True)

    def _drain_rows(sem, st):
        pltpu.make_async_copy(h_hbm.at[pidx.at[0, 0]], rows_v.at[st],
                              sem).wait()

    for j in range(DQ):
        _fire_idx(j, j)
    for j in range(DQ):
        _drain_idx(j)
        _fire_g(j)
    for j in range(DQ, 2 * DQ):
        _fire_idx(j, j)

    @pl.loop(0, NCH)
    def _grp(x):
        st = lax.rem(x, NSET)
        sg = lax.rem(x + DQ, NSET)
        sf = lax.rem(x + 2 * DQ, NSET)

        @pl.when(x > DQ - 1)
        def _():
            _drain_rows(ssem, sf)

        @pl.when(x + 2 * DQ < NCH)
        def _():
            _fire_idx(x + 2 * DQ, sf)

        @pl.when(x + DQ < NCH)
        def _():
            _drain_idx(sg)
            _fire_g(sg)

        _drain_rows(gsem, st)
        _fire_s(st)

    for j in range(DQ):
        _drain_rows(ssem, lax.rem(jnp.int32(NCH - DQ + j), NSET))
    plsc.subcore_barrier()

    for t in range(NRB):
        r = row0 + t * RB
        pltpu.sync_copy(agg_sh.at[pl.ds(r, RB)], agg_hbm.at[c, pl.ds(r, RB)])


def _norm_body(x_ref, degs_ref, degd_ref, hs_ref, ns_ref, nd_ref):
    deg_out = jnp.sum(degs_ref[...], axis=1, keepdims=True)
    deg_in = jnp.sum(degd_ref[...], axis=1, keepdims=True)
    ns = jnp.where(deg_out > 0, lax.rsqrt(jnp.maximum(deg_out, 1.0)), 0.0)
    nd = jnp.where(deg_in > 0, lax.rsqrt(jnp.maximum(deg_in, 1.0)), 0.0)
    ns_ref[...] = ns
    nd_ref[...] = nd
    hs_ref[...] = (x_ref[...] * ns).astype(BD)


_norm_call = pl.pallas_call(
    _norm_body,
    out_shape=(jax.ShapeDtypeStruct((N, D), BD),
               jax.ShapeDtypeStruct((N, 1), jnp.float32),
               jax.ShapeDtypeStruct((N, 1), jnp.float32)),
)


def _layer_body(relu_and_prescale, aggp_ref, nd_ref, ns_ref, w_ref, b_ref,
                out_ref):
    agg = (aggp_ref[0].astype(jnp.float32) +
           aggp_ref[1].astype(jnp.float32))
    h = agg * nd_ref[...]
    y = jnp.dot(h, w_ref[...], preferred_element_type=jnp.float32)
    y = y + b_ref[...]
    if relu_and_prescale:
        y = jnp.maximum(y, 0.0) * ns_ref[...]
        out_ref[...] = y.astype(BD)
    else:
        out_ref[...] = y


_layer_mid = pl.pallas_call(
    functools.partial(_layer_body, True),
    out_shape=jax.ShapeDtypeStruct((N, D), BD),
)
_layer_last = pl.pallas_call(
    functools.partial(_layer_body, False),
    out_shape=jax.ShapeDtypeStruct((N, D), jnp.float32),
)


def kernel(x, edge_index, W1, b1, g1, be1, W2, b2, g2, be2, W3, b3):
    eidx = edge_index.reshape(2, NW, NCH, CH).transpose(1, 2, 0, 3)
    src_d = edge_index[0].reshape(NW, EP // L, L)
    dst_d = edge_index[1].reshape(NW, EP // L, L)

    degs_p, degd_p = _deg_kernel(src_d, dst_d)
    hs, ns, nd = _norm_call(x, degs_p.T, degd_p.T)

    sc = 1.0 / jnp.sqrt(jnp.float32(1.0) + BN_EPS)
    Wf1 = W1 * (g1 * sc)[None, :]
    bf1 = (b1 * g1 * sc + be1).reshape(1, D)
    Wf2 = W2 * (g2 * sc)[None, :]
    bf2 = (b2 * g2 * sc + be2).reshape(1, D)
    bf3 = b3.reshape(1, D)

    aggp = _spmm_kernel(hs, eidx)
    hs = _layer_mid(aggp, nd, ns, Wf1, bf1)
    aggp = _spmm_kernel(hs, eidx)
    hs = _layer_mid(aggp, nd, ns, Wf2, bf2)
    aggp = _spmm_kernel(hs, eidx)
    out = _layer_last(aggp, nd, ns, W3, bf3)
    return out


# speedup vs baseline: 1.2309x; 1.0006x over previous
import functools

import jax
import jax.numpy as jnp
from jax import lax
from jax.experimental import pallas as pl
from jax.experimental.pallas import tpu as pltpu
from jax.experimental.pallas import tpu_sc as plsc

N = 10000
E = 320000
D = 128
BN_EPS = 1e-5

NC = 2
NS = 16
NW = NC * NS
L = 16

EP = E // NW
CH = 100
NCH = EP // CH
RT = N // NS
RB = 125
NRB = RT // RB
DQ = 3
NSET = 3 * DQ
BD = jnp.bfloat16
LB = 32

_mesh = functools.partial(
    plsc.VectorSubcoreMesh, core_axis_name="c", subcore_axis_name="s",
    num_cores=NC, num_subcores=NS)
_sc_params = pltpu.CompilerParams(
    needs_layout_passes=False, use_tc_tiling_on_sc=False,
    skip_device_barrier=True)
_tc_params = pltpu.CompilerParams(skip_device_barrier=True)


@functools.partial(
    pl.kernel,
    out_type=(jax.ShapeDtypeStruct((NW, N), jnp.float32),
              jax.ShapeDtypeStruct((NW, N), jnp.float32)),
    mesh=_mesh(),
    compiler_params=_sc_params,
    scratch_types=[
        pltpu.VMEM((EP // L, L), jnp.int32),
        pltpu.VMEM((EP // L, L), jnp.int32),
        pltpu.VMEM((N,), jnp.float32),
        pltpu.VMEM((N,), jnp.float32),
    ],
)
def _deg_kernel(src_hbm, dst_hbm, degs_hbm, degd_hbm,
                src_v, dst_v, degs_v, degd_v):
    c = lax.axis_index("c")
    s = lax.axis_index("s")
    wid = s * NC + c
    pltpu.sync_copy(src_hbm.at[wid], src_v)
    pltpu.sync_copy(dst_hbm.at[wid], dst_v)

    z = jnp.zeros((L,), jnp.float32)

    @pl.loop(0, N // L)
    def _zero(i):
        degs_v[pl.ds(i * L, L)] = z
        degd_v[pl.ds(i * L, L)] = z

    ones = jnp.ones((L,), jnp.float32)

    @pl.loop(0, EP // L)
    def _acc(j):
        plsc.addupdate_scatter(degs_v, [src_v[j]], ones)
        plsc.addupdate_scatter(degd_v, [dst_v[j]], ones)

    pltpu.sync_copy(degs_v, degs_hbm.at[wid])
    pltpu.sync_copy(degd_v, degd_hbm.at[wid])


@functools.partial(
    pl.kernel,
    out_type=jax.ShapeDtypeStruct((NC, N, D), BD),
    mesh=_mesh(),
    compiler_params=_sc_params,
    scratch_types=[
        pltpu.VMEM((NSET, 2, CH), jnp.int32),
        pltpu.VMEM((NSET, CH, D), BD),
        pltpu.VMEM_SHARED((N, D), BD),
        pltpu.SemaphoreType.DMA,
        pltpu.SemaphoreType.DMA,
        pltpu.SemaphoreType.DMA,
    ],
)
def _spmm_kernel(h_hbm, eidx_hbm, agg_hbm,
                 pidx, rows_v, agg_sh, gsem, ssem, isem):
    c = lax.axis_index("c")
    s = lax.axis_index("s")
    wid = s * NC + c

    z = jnp.zeros((LB,), BD)

    @pl.loop(0, CH)
    def _zb(j):
        for k in range(D // LB):
            rows_v[0, j, pl.ds(k * LB, LB)] = z

    row0 = s * RT
    nz = RT // CH
    for t in range(nz):
        pltpu.async_copy(rows_v.at[0], agg_sh.at[pl.ds(row0 + t * CH, CH)],
                         ssem)
    rz = RT - nz * CH
    pltpu.async_copy(rows_v.at[0, pl.ds(0, rz)],
                     agg_sh.at[pl.ds(row0 + nz * CH, rz)], ssem)
    for t in range(nz):
        pltpu.make_async_copy(rows_v.at[0], agg_sh.at[pl.ds(0, CH)],
                              ssem).wait()
    pltpu.make_async_copy(rows_v.at[0, pl.ds(0, rz)],
                          agg_sh.at[pl.ds(0, rz)], ssem).wait()
    plsc.subcore_barrier()

    def _fire_idx(x, st):
        pltpu.async_copy(eidx_hbm.at[wid, x], pidx.at[st], isem)

    def _drain_idx(st):
        pltpu.make_async_copy(eidx_hbm.at[wid, 0], pidx.at[st], isem).wait()

    def _fire_g(st):
        pltpu.async_copy(h_hbm.at[pidx.at[st, 0]], rows_v.at[st], gsem)

    def _fire_s(st):
        pltpu.async_copy(rows_v.at[st], agg_sh.at[pidx.at[st, 1]], ssem,
                         add=True)

    def _drain_rows(sem, st):
        pltpu.make_async_copy(h_hbm.at[pidx.at[0, 0]], rows_v.at[st],
                              sem).wait()

    for j in range(DQ):
        _fire_idx(j, j)
    for j in range(DQ):
        _drain_idx(j)
        _fire_g(j)
    for j in range(DQ, 2 * DQ):
        _fire_idx(j, j)

    @pl.loop(0, NCH)
    def _grp(x):
        st = lax.rem(x, NSET)
        sg = lax.rem(x + DQ, NSET)
        sf = lax.rem(x + 2 * DQ, NSET)

        @pl.when(x > DQ - 1)
        def _():
            _drain_rows(ssem, sf)

        @pl.when(x + 2 * DQ < NCH)
        def _():
            _fire_idx(x + 2 * DQ, sf)

        @pl.when(x + DQ < NCH)
        def _():
            _drain_idx(sg)
            _fire_g(sg)

        _drain_rows(gsem, st)
        _fire_s(st)

    for j in range(DQ):
        _drain_rows(ssem, lax.rem(jnp.int32(NCH - DQ + j), NSET))
    plsc.subcore_barrier()

    for t in range(NRB):
        r = row0 + t * RB
        pltpu.sync_copy(agg_sh.at[pl.ds(r, RB)], agg_hbm.at[c, pl.ds(r, RB)])


def _norm_body(x_ref, degs_ref, degd_ref, hs_ref, ns_ref, nd_ref):
    deg_out = jnp.sum(degs_ref[...], axis=1, keepdims=True)
    deg_in = jnp.sum(degd_ref[...], axis=1, keepdims=True)
    ns = jnp.where(deg_out > 0, lax.rsqrt(jnp.maximum(deg_out, 1.0)), 0.0)
    nd = jnp.where(deg_in > 0, lax.rsqrt(jnp.maximum(deg_in, 1.0)), 0.0)
    ns_ref[...] = ns
    nd_ref[...] = nd
    hs_ref[...] = (x_ref[...] * ns).astype(BD)


_norm_call = pl.pallas_call(
    _norm_body,
    out_shape=(jax.ShapeDtypeStruct((N, D), BD),
               jax.ShapeDtypeStruct((N, 1), jnp.float32),
               jax.ShapeDtypeStruct((N, 1), jnp.float32)),
    compiler_params=_tc_params,
)


def _layer_body(relu_and_prescale, aggp_ref, nd_ref, ns_ref, w_ref, b_ref,
                out_ref):
    agg = (aggp_ref[0].astype(jnp.float32) +
           aggp_ref[1].astype(jnp.float32))
    h = agg * nd_ref[...]
    y = jnp.dot(h, w_ref[...], preferred_element_type=jnp.float32)
    y = y + b_ref[...]
    if relu_and_prescale:
        y = jnp.maximum(y, 0.0) * ns_ref[...]
        out_ref[...] = y.astype(BD)
    else:
        out_ref[...] = y


_layer_mid = pl.pallas_call(
    functools.partial(_layer_body, True),
    out_shape=jax.ShapeDtypeStruct((N, D), BD),
    compiler_params=_tc_params,
)
_layer_last = pl.pallas_call(
    functools.partial(_layer_body, False),
    out_shape=jax.ShapeDtypeStruct((N, D), jnp.float32),
    compiler_params=_tc_params,
)


def kernel(x, edge_index, W1, b1, g1, be1, W2, b2, g2, be2, W3, b3):
    eidx = edge_index.reshape(2, NW, NCH, CH).transpose(1, 2, 0, 3)
    src_d = edge_index[0].reshape(NW, EP // L, L)
    dst_d = edge_index[1].reshape(NW, EP // L, L)

    degs_p, degd_p = _deg_kernel(src_d, dst_d)
    hs, ns, nd = _norm_call(x, degs_p.T, degd_p.T)

    sc = 1.0 / jnp.sqrt(jnp.float32(1.0) + BN_EPS)
    Wf1 = W1 * (g1 * sc)[None, :]
    bf1 = (b1 * g1 * sc + be1).reshape(1, D)
    Wf2 = W2 * (g2 * sc)[None, :]
    bf2 = (b2 * g2 * sc + be2).reshape(1, D)
    bf3 = b3.reshape(1, D)

    aggp = _spmm_kernel(hs, eidx)
    hs = _layer_mid(aggp, nd, ns, Wf1, bf1)
    aggp = _spmm_kernel(hs, eidx)
    hs = _layer_mid(aggp, nd, ns, Wf2, bf2)
    aggp = _spmm_kernel(hs, eidx)
    out = _layer_last(aggp, nd, ns, W3, bf3)
    return out


# speedup vs baseline: 1.2410x; 1.0082x over previous
import functools

import jax
import jax.numpy as jnp
from jax import lax
from jax.experimental import pallas as pl
from jax.experimental.pallas import tpu as pltpu
from jax.experimental.pallas import tpu_sc as plsc

N = 10000
E = 320000
D = 128
BN_EPS = 1e-5

NC = 2
NS = 16
NW = NC * NS
L = 16

EP = E // NW
CH = 100
NCH = EP // CH
RT = N // NS
RB = 125
NRB = RT // RB
DQ = 3
NSET = 3 * DQ
BD = jnp.bfloat16
LB = 32

_mesh = functools.partial(
    plsc.VectorSubcoreMesh, core_axis_name="c", subcore_axis_name="s",
    num_cores=NC, num_subcores=NS)
_sc_params = pltpu.CompilerParams(
    needs_layout_passes=False, use_tc_tiling_on_sc=False)


@functools.partial(
    pl.kernel,
    out_type=(jax.ShapeDtypeStruct((NW, N), jnp.float32),
              jax.ShapeDtypeStruct((NW, N), jnp.float32)),
    mesh=_mesh(),
    compiler_params=_sc_params,
    scratch_types=[
        pltpu.VMEM((EP // L, L), jnp.int32),
        pltpu.VMEM((EP // L, L), jnp.int32),
        pltpu.VMEM((N,), jnp.float32),
        pltpu.VMEM((N,), jnp.float32),
    ],
)
def _deg_kernel(src_hbm, dst_hbm, degs_hbm, degd_hbm,
                src_v, dst_v, degs_v, degd_v):
    c = lax.axis_index("c")
    s = lax.axis_index("s")
    wid = s * NC + c
    pltpu.sync_copy(src_hbm.at[wid], src_v)
    pltpu.sync_copy(dst_hbm.at[wid], dst_v)

    z = jnp.zeros((L,), jnp.float32)

    @pl.loop(0, N // L)
    def _zero(i):
        degs_v[pl.ds(i * L, L)] = z
        degd_v[pl.ds(i * L, L)] = z

    ones = jnp.ones((L,), jnp.float32)

    @pl.loop(0, EP // L)
    def _acc(j):
        plsc.addupdate_scatter(degs_v, [src_v[j]], ones)
        plsc.addupdate_scatter(degd_v, [dst_v[j]], ones)

    pltpu.sync_copy(degs_v, degs_hbm.at[wid])
    pltpu.sync_copy(degd_v, degd_hbm.at[wid])


@functools.partial(
    pl.kernel,
    out_type=jax.ShapeDtypeStruct((NC, N, D), BD),
    mesh=_mesh(),
    compiler_params=_sc_params,
    scratch_types=[
        pltpu.VMEM((NSET, CH), jnp.int32),
        pltpu.VMEM((NSET, CH), jnp.int32),
        pltpu.VMEM((NSET, CH, D), BD),
        pltpu.VMEM_SHARED((N, D), BD),
        pltpu.SemaphoreType.DMA,
        pltpu.SemaphoreType.DMA,
        pltpu.SemaphoreType.DMA,
    ],
)
def _spmm_kernel(h_hbm, src_hbm, dst_hbm, agg_hbm,
                 sidx, didx, rows_v, agg_sh, gsem, ssem, isem):
    c = lax.axis_index("c")
    s = lax.axis_index("s")
    wid = s * NC + c

    z = jnp.zeros((LB,), BD)

    @pl.loop(0, CH)
    def _zb(j):
        for k in range(D // LB):
            rows_v[0, j, pl.ds(k * LB, LB)] = z

    row0 = s * RT
    nz = RT // CH
    for t in range(nz):
        pltpu.async_copy(rows_v.at[0], agg_sh.at[pl.ds(row0 + t * CH, CH)],
                         ssem)
    rz = RT - nz * CH
    pltpu.async_copy(rows_v.at[0, pl.ds(0, rz)],
                     agg_sh.at[pl.ds(row0 + nz * CH, rz)], ssem)
    for t in range(nz):
        pltpu.make_async_copy(rows_v.at[0], agg_sh.at[pl.ds(0, CH)],
                              ssem).wait()
    pltpu.make_async_copy(rows_v.at[0, pl.ds(0, rz)],
                          agg_sh.at[pl.ds(0, rz)], ssem).wait()
    plsc.subcore_barrier()

    def _fire_idx(x, st):
        pltpu.async_copy(src_hbm.at[wid, x], sidx.at[st], isem)
        pltpu.async_copy(dst_hbm.at[wid, x], didx.at[st], isem)

    def _drain_idx(st):
        pltpu.make_async_copy(src_hbm.at[wid, 0], sidx.at[st], isem).wait()
        pltpu.make_async_copy(dst_hbm.at[wid, 0], didx.at[st], isem).wait()

    def _fire_g(st):
        pltpu.async_copy(h_hbm.at[sidx.at[st]], rows_v.at[st], gsem)

    def _fire_s(st):
        pltpu.async_copy(rows_v.at[st], agg_sh.at[didx.at[st]], ssem,
                         add=True)

    def _drain_rows(sem, st):
        pltpu.make_async_copy(h_hbm.at[sidx.at[0]], rows_v.at[st],
                              sem).wait()

    for j in range(DQ):
        _fire_idx(j, j)
    for j in range(DQ):
        _drain_idx(j)
        _fire_g(j)
    for j in range(DQ, 2 * DQ):
        _fire_idx(j, j)

    @pl.loop(0, NCH)
    def _grp(x):
        st = lax.rem(x, NSET)
        sg = lax.rem(x + DQ, NSET)
        sf = lax.rem(x + 2 * DQ, NSET)

        @pl.when(x > DQ - 1)
        def _():
            _drain_rows(ssem, sf)

        @pl.when(x + 2 * DQ < NCH)
        def _():
            _fire_idx(x + 2 * DQ, sf)

        @pl.when(x + DQ < NCH)
        def _():
            _drain_idx(sg)
            _fire_g(sg)

        _drain_rows(gsem, st)
        _fire_s(st)

    for j in range(DQ):
        _drain_rows(ssem, lax.rem(jnp.int32(NCH - DQ + j), NSET))
    plsc.subcore_barrier()

    for t in range(NRB):
        r = row0 + t * RB
        pltpu.sync_copy(agg_sh.at[pl.ds(r, RB)], agg_hbm.at[c, pl.ds(r, RB)])


def _norm_body(x_ref, degs_ref, degd_ref, hs_ref, ns_ref, nd_ref):
    deg_out = jnp.sum(degs_ref[...], axis=1, keepdims=True)
    deg_in = jnp.sum(degd_ref[...], axis=1, keepdims=True)
    ns = jnp.where(deg_out > 0, lax.rsqrt(jnp.maximum(deg_out, 1.0)), 0.0)
    nd = jnp.where(deg_in > 0, lax.rsqrt(jnp.maximum(deg_in, 1.0)), 0.0)
    ns_ref[...] = ns
    nd_ref[...] = nd
    hs_ref[...] = (x_ref[...] * ns).astype(BD)


_norm_call = pl.pallas_call(
    _norm_body,
    out_shape=(jax.ShapeDtypeStruct((N, D), BD),
               jax.ShapeDtypeStruct((N, 1), jnp.float32),
               jax.ShapeDtypeStruct((N, 1), jnp.float32)),
)


def _layer_body(relu_and_prescale, aggp_ref, nd_ref, ns_ref, w_ref, b_ref,
                out_ref):
    agg = (aggp_ref[0].astype(jnp.float32) +
           aggp_ref[1].astype(jnp.float32))
    h = agg * nd_ref[...]
    y = jnp.dot(h, w_ref[...], preferred_element_type=jnp.float32)
    y = y + b_ref[...]
    if relu_and_prescale:
        y = jnp.maximum(y, 0.0) * ns_ref[...]
        out_ref[...] = y.astype(BD)
    else:
        out_ref[...] = y


_layer_mid = pl.pallas_call(
    functools.partial(_layer_body, True),
    out_shape=jax.ShapeDtypeStruct((N, D), BD),
)
_layer_last = pl.pallas_call(
    functools.partial(_layer_body, False),
    out_shape=jax.ShapeDtypeStruct((N, D), jnp.float32),
)


def kernel(x, edge_index, W1, b1, g1, be1, W2, b2, g2, be2, W3, b3):
    src2 = edge_index[0].reshape(NW, NCH, CH)
    dst2 = edge_index[1].reshape(NW, NCH, CH)
    src_d = edge_index[0].reshape(NW, EP // L, L)
    dst_d = edge_index[1].reshape(NW, EP // L, L)

    degs_p, degd_p = _deg_kernel(src_d, dst_d)
    hs, ns, nd = _norm_call(x, degs_p.T, degd_p.T)

    sc = 1.0 / jnp.sqrt(jnp.float32(1.0) + BN_EPS)
    Wf1 = W1 * (g1 * sc)[None, :]
    bf1 = (b1 * g1 * sc + be1).reshape(1, D)
    Wf2 = W2 * (g2 * sc)[None, :]
    bf2 = (b2 * g2 * sc + be2).reshape(1, D)
    bf3 = b3.reshape(1, D)

    aggp = _spmm_kernel(hs, src2, dst2)
    hs = _layer_mid(aggp, nd, ns, Wf1, bf1)
    aggp = _spmm_kernel(hs, src2, dst2)
    hs = _layer_mid(aggp, nd, ns, Wf2, bf2)
    aggp = _spmm_kernel(hs, src2, dst2)
    out = _layer_last(aggp, nd, ns, W3, bf3)
    return out
